# baseline (speedup 1.0000x reference)
import jax
import jax.numpy as jnp
from jax import lax
from jax.experimental import pallas as pl
from jax.experimental.pallas import tpu as pltpu

_BINS = 256
_INV_S2 = 1.0e4
_BN_EPS = 1e-5
_HALO = 128


def _hist_body(x_ref, o_ref):
    centers = lax.broadcasted_iota(jnp.int32, (_BINS, 128), 0).astype(
        jnp.float32
    ) * (1.0 / 255.0)

    def body(i, acc):
        xt = x_ref[0, pl.ds(i * 8, 8), :]
        for s in range(8):
            d = xt[s : s + 1, :] - centers
            acc = acc + jnp.exp(d * d * (-_INV_S2))
        return acc

    acc = lax.fori_loop(0, 49, body, jnp.zeros((_BINS, 128), jnp.float32))

    ident = jnp.where(
        lax.broadcasted_iota(jnp.int32, (_BINS, _BINS), 0)
        == lax.broadcasted_iota(jnp.int32, (_BINS, _BINS), 1),
        1.0,
        0.0,
    )
    acc_t = lax.dot_general(
        acc, ident, (((0,), (0,)), ((), ())), preferred_element_type=jnp.float32
    )
    hist = jnp.sum(acc_t, axis=0, keepdims=True)
    total = jnp.sum(hist, axis=1, keepdims=True)
    pdf = hist / (total + 1e-6)
    upper = jnp.where(
        lax.broadcasted_iota(jnp.int32, (_BINS, _BINS), 0)
        <= lax.broadcasted_iota(jnp.int32, (_BINS, _BINS), 1),
        1.0,
        0.0,
    )
    cdf = jnp.dot(pdf, upper, preferred_element_type=jnp.float32)
    o_ref[0] = cdf


def _soft_cdf(x):
    B, C, H, W = x.shape
    xr = x.reshape(B * C, 392, 128)
    out = pl.pallas_call(
        _hist_body,
        grid=(B * C,),
        in_specs=[pl.BlockSpec((1, 392, 128), lambda i: (i, 0, 0))],
        out_specs=pl.BlockSpec((1, 1, _BINS), lambda i: (i, 0, 0)),
        out_shape=jax.ShapeDtypeStruct((B * C, 1, _BINS), jnp.float32),
        compiler_params=pltpu.CompilerParams(
            dimension_semantics=("parallel",),
        ),
    )(xr)
    return out.reshape(B, C * _BINS)


def _pad_mask(shape):
    ii = lax.broadcasted_iota(jnp.int32, shape, 0)
    return ((ii % 64) < 56) & ((ii % 8) < 7)


def _finish(acc, res, act, out_ref, R):
    if res is not None:
        acc = acc + res
    if act:
        acc = jnp.maximum(acc, 0.0)
    out_ref[_HALO : _HALO + R, :] = jnp.where(_pad_mask(acc.shape), acc, 0.0)


def _conv_s1(in_ref, out_ref, M, w_ref, b_ref, act=True, res_ref=None):
    R = M * M * 64
    blocks = []
    for ay in range(M):
        for ax in range(M):
            taps = []
            for dy in (-1, 0, 1):
                for dx in (-1, 0, 1):
                    t = ay + dy
                    by = t % M
                    cy = (t - by) // M
                    u = ax + dx
                    bx = u % M
                    cx = (u - bx) // M
                    base = _HALO + (by * M + bx) * 64 + cy * 8 + cx
                    taps.append(in_ref[base : base + 64, :])
            blocks.append(jnp.concatenate(taps, axis=1).astype(jnp.bfloat16))
    lhs = jnp.concatenate(blocks, axis=0)
    acc = jnp.dot(lhs, w_ref[...], preferred_element_type=jnp.float32) + b_ref[...]
    res = None if res_ref is None else res_ref[_HALO : _HALO + R, :]
    _finish(acc, res, act, out_ref, R)


def _conv_s2(in_ref, out_ref, m_in, w_ref, b_ref, taps3, act=True, res_ref=None):
    m_out = m_in // 2
    R = m_out * m_out * 64
    tapset = (
        [(dy, dx) for dy in (-1, 0, 1) for dx in (-1, 0, 1)] if taps3 else [(0, 0)]
    )
    blocks = []
    for ay in range(m_out):
        for ax in range(m_out):
            taps = []
            for dy, dx in tapset:
                t = 2 * ay + dy
                by = t % m_in
                cy = (t - by) // m_in
                u = 2 * ax + dx
                bx = u % m_in
                cx = (u - bx) // m_in
                base = _HALO + (by * m_in + bx) * 64 + cy * 8 + cx
                taps.append(in_ref[base : base + 64, :])
            blocks.append(jnp.concatenate(taps, axis=1).astype(jnp.bfloat16))
    lhs = jnp.concatenate(blocks, axis=0)
    acc = jnp.dot(lhs, w_ref[...], preferred_element_type=jnp.float32) + b_ref[...]
    res = None if res_ref is None else res_ref[_HALO : _HALO + R, :]
    _finish(acc, res, act, out_ref, R)


def _maxpool(in_ref, out_ref):
    m_in, m_out = 16, 8
    R = m_out * m_out * 64
    blocks = []
    for ay in range(m_out):
        for ax in range(m_out):
            cur = None
            for dy in (-1, 0, 1):
                for dx in (-1, 0, 1):
                    t = 2 * ay + dy
                    by = t % m_in
                    cy = (t - by) // m_in
                    u = 2 * ax + dx
                    bx = u % m_in
                    cx = (u - bx) // m_in
                    base = _HALO + (by * m_in + bx) * 64 + cy * 8 + cx
                    v = in_ref[base : base + 64, :]
                    cur = v if cur is None else jnp.maximum(cur, v)
            blocks.append(cur)
    acc = jnp.concatenate(blocks, axis=0)
    _finish(acc, None, False, out_ref, R)


def _resnet_body(pt_ref, *refs):
    ws = refs[:20]
    bs = refs[20:40]
    o_ref = refs[40]
    (sA, s1a, s1b, s1c, s2a, s2b, s2c, s3a, s3b, s3c, s4a, s4b, s4c) = refs[41:]

    for ref in (sA, s1a, s1b, s1c, s2a, s2b, s2c, s3a, s3b, s3c, s4a, s4b, s4c):
        n = ref.shape[0]
        c = ref.shape[1]
        ref[0:_HALO, :] = jnp.zeros((_HALO, c), jnp.float32)
        ref[n - _HALO : n, :] = jnp.zeros((_HALO, c), jnp.float32)

    acc = (
        jnp.dot(pt_ref[0], ws[0][...], preferred_element_type=jnp.float32)
        + bs[0][...]
    )
    _finish(acc, None, True, sA, 16384)

    _maxpool(sA, s1a)

    _conv_s1(s1a, s1b, 8, ws[1], bs[1])
    _conv_s1(s1b, s1c, 8, ws[2], bs[2], res_ref=s1a)
    _conv_s1(s1c, s1b, 8, ws[3], bs[3])
    _conv_s1(s1b, s1a, 8, ws[4], bs[4], res_ref=s1c)

    _conv_s2(s1a, s2c, 8, ws[5], bs[5], taps3=False, act=False)
    _conv_s2(s1a, s2a, 8, ws[6], bs[6], taps3=True)
    _conv_s1(s2a, s2b, 4, ws[7], bs[7], res_ref=s2c)
    _conv_s1(s2b, s2a, 4, ws[8], bs[8])
    _conv_s1(s2a, s2c, 4, ws[9], bs[9], res_ref=s2b)

    _conv_s2(s2c, s3c, 4, ws[10], bs[10], taps3=False, act=False)
    _conv_s2(s2c, s3a, 4, ws[11], bs[11], taps3=True)
    _conv_s1(s3a, s3b, 2, ws[12], bs[12], res_ref=s3c)
    _conv_s1(s3b, s3a, 2, ws[13], bs[13])
    _conv_s1(s3a, s3c, 2, ws[14], bs[14], res_ref=s3b)

    _conv_s2(s3c, s4c, 2, ws[15], bs[15], taps3=False, act=False)
    _conv_s2(s3c, s4a, 2, ws[16], bs[16], taps3=True)
    _conv_s1(s4a, s4b, 1, ws[17], bs[17], res_ref=s4c)
    _conv_s1(s4b, s4a, 1, ws[18], bs[18])
    _conv_s1(s4a, s4c, 1, ws[19], bs[19], res_ref=s4b)

    data = s4c[_HALO : _HALO + 64, :]
    o_ref[0] = jnp.sum(data, axis=0, keepdims=True) * (1.0 / 49.0)


def _fold(w, bn):
    s = bn["g"] * lax.rsqrt(bn["v"] + _BN_EPS)
    t = bn["b"] - bn["m"] * s
    return w * s[:, None, None, None], t


def _w3(w):
    return jnp.transpose(w, (2, 3, 1, 0)).reshape(9 * w.shape[1], w.shape[0])


def _phase_stack_patches(p):
    B = p.shape[0]
    p = jnp.transpose(p, (0, 2, 3, 1))
    p = p.reshape(B, 7, 16, 7, 16, 147)
    p = jnp.transpose(p, (0, 2, 4, 1, 3, 5))
    p = jnp.pad(p, ((0, 0), (0, 0), (0, 0), (0, 1), (0, 1), (0, 0)))
    return p.reshape(B, 16384, 147)


def _prep_weights(params):
    ws, bs = [], []

    w, t = _fold(params["conv1"], params["bn1"])
    ws.append(w.reshape(64, 147).T)
    bs.append(t)

    for blocks in params["layers"]:
        for blk in blocks:
            if "down" in blk:
                wd, td = _fold(blk["down"], blk["dbn"])
                wd3 = wd[:, :, 0, 0].T
                w1, t1 = _fold(blk["conv1"], blk["bn1"])
                w2, t2 = _fold(blk["conv2"], blk["bn2"])
                ws.extend([wd3, _w3(w1), _w3(w2)])
                bs.extend([td, t1, t2])
            else:
                w1, t1 = _fold(blk["conv1"], blk["bn1"])
                w2, t2 = _fold(blk["conv2"], blk["bn2"])
                ws.extend([_w3(w1), _w3(w2)])
                bs.extend([t1, t2])
    ws = [w.astype(jnp.bfloat16) for w in ws]
    bs = [b.reshape(1, -1).astype(jnp.float32) for b in bs]
    return ws, bs


def _resnet(x, params):
    B = x.shape[0]
    patches = lax.conv_general_dilated_patches(
        x, (7, 7), (2, 2), [(3, 3), (3, 3)]
    )
    pt = _phase_stack_patches(patches).astype(jnp.bfloat16)
    ws, bs = _prep_weights(params)

    wspecs = [
        pl.BlockSpec(w.shape, (lambda b: (0, 0))) for w in ws
    ] + [pl.BlockSpec(b.shape, (lambda b: (0, 0))) for b in bs]

    scratch = [pltpu.VMEM((2 * _HALO + 16384, 64), jnp.float32)]
    for R, C in (
        (4096, 64), (4096, 64), (4096, 64),
        (1024, 128), (1024, 128), (1024, 128),
        (256, 256), (256, 256), (256, 256),
        (64, 512), (64, 512), (64, 512),
    ):
        scratch.append(pltpu.VMEM((2 * _HALO + R, C), jnp.float32))

    out = pl.pallas_call(
        _resnet_body,
        grid=(B,),
        in_specs=[pl.BlockSpec((1, 16384, 147), lambda b: (b, 0, 0))] + wspecs,
        out_specs=pl.BlockSpec((1, 1, 512), lambda b: (b, 0, 0)),
        out_shape=jax.ShapeDtypeStruct((B, 1, 512), jnp.float32),
        scratch_shapes=scratch,
        compiler_params=pltpu.CompilerParams(
            dimension_semantics=("parallel",),
            vmem_limit_bytes=100 * 1024 * 1024,
        ),
    )(pt, *ws, *bs)
    return out.reshape(B, 512)


def kernel(x, params):
    cdf = _soft_cdf(x)
    spatial = _resnet(x, params)
    return jnp.concatenate([cdf, spatial], axis=1)


# speedup vs baseline: 1.0351x; 1.0351x over previous
import jax
import jax.numpy as jnp
from jax import lax
from jax.experimental import pallas as pl
from jax.experimental.pallas import tpu as pltpu

_BINS = 256
_INV_S2 = 1.0e4
_BN_EPS = 1e-5
_HALO = 16


def _one_hist(x_ref, ch):
    centers = lax.broadcasted_iota(jnp.int32, (_BINS, 128), 0).astype(
        jnp.float32
    ) * (1.0 / 255.0)

    def body(i, acc):
        xt = x_ref[0, ch, pl.ds(i * 8, 8), :]
        for s in range(8):
            d = xt[s : s + 1, :] - centers
            acc = acc + jnp.exp(d * d * (-_INV_S2))
        return acc

    acc = lax.fori_loop(0, 49, body, jnp.zeros((_BINS, 128), jnp.float32))

    ident = jnp.where(
        lax.broadcasted_iota(jnp.int32, (_BINS, _BINS), 0)
        == lax.broadcasted_iota(jnp.int32, (_BINS, _BINS), 1),
        1.0,
        0.0,
    )
    acc_t = lax.dot_general(
        acc, ident, (((0,), (0,)), ((), ())), preferred_element_type=jnp.float32
    )
    hist = jnp.sum(acc_t, axis=0, keepdims=True)
    total = jnp.sum(hist, axis=1, keepdims=True)
    pdf = hist / (total + 1e-6)
    upper = jnp.where(
        lax.broadcasted_iota(jnp.int32, (_BINS, _BINS), 0)
        <= lax.broadcasted_iota(jnp.int32, (_BINS, _BINS), 1),
        1.0,
        0.0,
    )
    return jnp.dot(pdf, upper, preferred_element_type=jnp.float32)


def _pad_mask(shape):
    ii = lax.broadcasted_iota(jnp.int32, shape, 0)
    return ((ii % 64) < 56) & ((ii % 8) < 7)


def _finish(acc, res, act, out_ref, R):
    if res is not None:
        acc = acc + res
    if act:
        acc = jnp.maximum(acc, 0.0)
    out_ref[_HALO : _HALO + R, :] = jnp.where(_pad_mask(acc.shape), acc, 0.0)


def _conv_s1(in_ref, out_ref, M, w_ref, b_ref, act=True, res_ref=None):
    R = M * M * 64
    blocks = []
    for ay in range(M):
        for ax in range(M):
            taps = []
            for dy in (-1, 0, 1):
                for dx in (-1, 0, 1):
                    t = ay + dy
                    by = t % M
                    cy = (t - by) // M
                    u = ax + dx
                    bx = u % M
                    cx = (u - bx) // M
                    base = _HALO + (by * M + bx) * 64 + cy * 8 + cx
                    taps.append(in_ref[base : base + 64, :])
            blocks.append(jnp.concatenate(taps, axis=1).astype(jnp.bfloat16))
    lhs = jnp.concatenate(blocks, axis=0)
    acc = jnp.dot(lhs, w_ref[...], preferred_element_type=jnp.float32) + b_ref[...]
    res = None if res_ref is None else res_ref[_HALO : _HALO + R, :]
    _finish(acc, res, act, out_ref, R)


def _conv_s2(in_ref, out_ref, m_in, w_ref, b_ref, taps3, act=True, res_ref=None):
    m_out = m_in // 2
    R = m_out * m_out * 64
    tapset = (
        [(dy, dx) for dy in (-1, 0, 1) for dx in (-1, 0, 1)] if taps3 else [(0, 0)]
    )
    blocks = []
    for ay in range(m_out):
        for ax in range(m_out):
            taps = []
            for dy, dx in tapset:
                t = 2 * ay + dy
                by = t % m_in
                cy = (t - by) // m_in
                u = 2 * ax + dx
                bx = u % m_in
                cx = (u - bx) // m_in
                base = _HALO + (by * m_in + bx) * 64 + cy * 8 + cx
                taps.append(in_ref[base : base + 64, :])
            blocks.append(jnp.concatenate(taps, axis=1).astype(jnp.bfloat16))
    lhs = jnp.concatenate(blocks, axis=0)
    acc = jnp.dot(lhs, w_ref[...], preferred_element_type=jnp.float32) + b_ref[...]
    res = None if res_ref is None else res_ref[_HALO : _HALO + R, :]
    _finish(acc, res, act, out_ref, R)


def _maxpool(in_ref, out_ref):
    m_in, m_out = 16, 8
    R = m_out * m_out * 64
    blocks = []
    for ay in range(m_out):
        for ax in range(m_out):
            cur = None
            for dy in (-1, 0, 1):
                for dx in (-1, 0, 1):
                    t = 2 * ay + dy
                    by = t % m_in
                    cy = (t - by) // m_in
                    u = 2 * ax + dx
                    bx = u % m_in
                    cx = (u - bx) // m_in
                    base = _HALO + (by * m_in + bx) * 64 + cy * 8 + cx
                    v = in_ref[base : base + 64, :]
                    cur = v if cur is None else jnp.maximum(cur, v)
            blocks.append(cur)
    acc = jnp.concatenate(blocks, axis=0)
    _finish(acc, None, False, out_ref, R)


def _resnet_body(pt_ref, x_ref, *refs):
    ws = refs[:20]
    bs = refs[20:40]
    o_ref = refs[40]
    (sA, s1a, s1b, s1c, s2a, s2b, s2c, s3a, s3b, s3c, s4a, s4b, s4c) = refs[41:]

    for ch in range(3):
        o_ref[0, :, ch * _BINS : (ch + 1) * _BINS] = _one_hist(x_ref, ch)

    for ref in (sA, s1a, s1b, s1c, s2a, s2b, s2c, s3a, s3b, s3c, s4a, s4b, s4c):
        n = ref.shape[0]
        c = ref.shape[1]
        ref[0:_HALO, :] = jnp.zeros((_HALO, c), jnp.float32)
        ref[n - _HALO : n, :] = jnp.zeros((_HALO, c), jnp.float32)

    acc = (
        jnp.dot(pt_ref[0], ws[0][...], preferred_element_type=jnp.float32)
        + bs[0][...]
    )
    _finish(acc, None, True, sA, 16384)

    _maxpool(sA, s1a)

    _conv_s1(s1a, s1b, 8, ws[1], bs[1])
    _conv_s1(s1b, s1c, 8, ws[2], bs[2], res_ref=s1a)
    _conv_s1(s1c, s1b, 8, ws[3], bs[3])
    _conv_s1(s1b, s1a, 8, ws[4], bs[4], res_ref=s1c)

    _conv_s2(s1a, s2c, 8, ws[5], bs[5], taps3=False, act=False)
    _conv_s2(s1a, s2a, 8, ws[6], bs[6], taps3=True)
    _conv_s1(s2a, s2b, 4, ws[7], bs[7], res_ref=s2c)
    _conv_s1(s2b, s2a, 4, ws[8], bs[8])
    _conv_s1(s2a, s2c, 4, ws[9], bs[9], res_ref=s2b)

    _conv_s2(s2c, s3c, 4, ws[10], bs[10], taps3=False, act=False)
    _conv_s2(s2c, s3a, 4, ws[11], bs[11], taps3=True)
    _conv_s1(s3a, s3b, 2, ws[12], bs[12], res_ref=s3c)
    _conv_s1(s3b, s3a, 2, ws[13], bs[13])
    _conv_s1(s3a, s3c, 2, ws[14], bs[14], res_ref=s3b)

    _conv_s2(s3c, s4c, 2, ws[15], bs[15], taps3=False, act=False)
    _conv_s2(s3c, s4a, 2, ws[16], bs[16], taps3=True)
    _conv_s1(s4a, s4b, 1, ws[17], bs[17], res_ref=s4c)
    _conv_s1(s4b, s4a, 1, ws[18], bs[18])
    _conv_s1(s4a, s4c, 1, ws[19], bs[19], res_ref=s4b)

    data = s4c[_HALO : _HALO + 64, :]
    o_ref[0, :, 768:1280] = jnp.sum(data, axis=0, keepdims=True) * (1.0 / 49.0)


def _fold(w, bn):
    s = bn["g"] * lax.rsqrt(bn["v"] + _BN_EPS)
    t = bn["b"] - bn["m"] * s
    return w * s[:, None, None, None], t


def _w3(w):
    return jnp.transpose(w, (2, 3, 1, 0)).reshape(9 * w.shape[1], w.shape[0])


def _phase_stack_patches(p):
    B = p.shape[0]
    p = jnp.transpose(p, (0, 2, 3, 1))
    p = p.reshape(B, 7, 16, 7, 16, 147)
    p = jnp.transpose(p, (0, 2, 4, 1, 3, 5))
    p = jnp.pad(p, ((0, 0), (0, 0), (0, 0), (0, 1), (0, 1), (0, 0)))
    return p.reshape(B, 16384, 147)


def _prep_weights(params):
    ws, bs = [], []

    w, t = _fold(params["conv1"], params["bn1"])
    ws.append(w.reshape(64, 147).T)
    bs.append(t)

    for blocks in params["layers"]:
        for blk in blocks:
            if "down" in blk:
                wd, td = _fold(blk["down"], blk["dbn"])
                wd3 = wd[:, :, 0, 0].T
                w1, t1 = _fold(blk["conv1"], blk["bn1"])
                w2, t2 = _fold(blk["conv2"], blk["bn2"])
                ws.extend([wd3, _w3(w1), _w3(w2)])
                bs.extend([td, t1, t2])
            else:
                w1, t1 = _fold(blk["conv1"], blk["bn1"])
                w2, t2 = _fold(blk["conv2"], blk["bn2"])
                ws.extend([_w3(w1), _w3(w2)])
                bs.extend([t1, t2])
    ws = [w.astype(jnp.bfloat16) for w in ws]
    bs = [b.reshape(1, -1).astype(jnp.float32) for b in bs]
    return ws, bs


def _resnet(x, params):
    B = x.shape[0]
    patches = lax.conv_general_dilated_patches(
        x.astype(jnp.bfloat16), (7, 7), (2, 2), [(3, 3), (3, 3)]
    )
    pt = _phase_stack_patches(patches)
    xr = x.reshape(B, 3, 392, 128)
    ws, bs = _prep_weights(params)

    wspecs = [
        pl.BlockSpec(w.shape, (lambda b: (0, 0))) for w in ws
    ] + [pl.BlockSpec(b.shape, (lambda b: (0, 0))) for b in bs]

    scratch = [pltpu.VMEM((2 * _HALO + 16384, 64), jnp.float32)]
    for R, C in (
        (4096, 64), (4096, 64), (4096, 64),
        (1024, 128), (1024, 128), (1024, 128),
        (256, 256), (256, 256), (256, 256),
        (64, 512), (64, 512), (64, 512),
    ):
        scratch.append(pltpu.VMEM((2 * _HALO + R, C), jnp.float32))

    out = pl.pallas_call(
        _resnet_body,
        grid=(B,),
        in_specs=[
            pl.BlockSpec((1, 16384, 147), lambda b: (b, 0, 0)),
            pl.BlockSpec((1, 3, 392, 128), lambda b: (b, 0, 0, 0)),
        ]
        + wspecs,
        out_specs=pl.BlockSpec((1, 1, 1280), lambda b: (b, 0, 0)),
        out_shape=jax.ShapeDtypeStruct((B, 1, 1280), jnp.float32),
        scratch_shapes=scratch,
        compiler_params=pltpu.CompilerParams(
            dimension_semantics=("arbitrary",),
            vmem_limit_bytes=100 * 1024 * 1024,
        ),
    )(pt, xr, *ws, *bs)
    return out.reshape(B, 1280)


def kernel(x, params):
    return _resnet(x, params)
